# Initial kernel scaffold; baseline (speedup 1.0000x reference)
import functools

import jax
import jax.numpy as jnp
from jax import lax
from jax.experimental import pallas as pl
from jax.experimental.pallas import tpu as pltpu
from jax.experimental.pallas import tpu_sc as plsc

N = 10000
E = 320000
IN_FEATS = 128
HIDDEN = 16
HEADS = 8
OUT_FEATS = 64

NC = 2
NS = 16
NW = NC * NS
L = 16

CHUNK = 128
CPT = -(-E // (NW * CHUNK))
E_PAD = NW * CPT * CHUNK

NP = 10240
D1 = 144
D2 = 80
BLK = 256

_mesh = plsc.VectorSubcoreMesh(core_axis_name="c", subcore_axis_name="s")


def _zero_rows(rows_v, width):
  z = jnp.zeros((L,), jnp.float32)

  @pl.loop(0, CHUNK)
  def _(r):
    for q in range(width // L):
      rows_v[r, pl.ds(q * L, L)] = z


def _edge_kernel(heads, hid, width, acol,
                 tab_hbm, adst_hbm, sidx_hbm, didx_hbm, out_hbm,
                 sidx_v, didx_v, rows_v, adst_v, acc_sh, sem_g, sem_a):
  c = lax.axis_index("c")
  s = lax.axis_index("s")
  wid = c * NS + s

  _zero_rows(rows_v, width)
  rpt = NP // NS // CHUNK
  for b in range(rpt):
    base = (s * rpt + b) * CHUNK
    pltpu.sync_copy(rows_v, acc_sh.at[pl.ds(base, CHUNK)])
  plsc.subcore_barrier()

  pltpu.sync_copy(sidx_hbm.at[wid], sidx_v)
  pltpu.sync_copy(didx_hbm.at[wid], didx_v)

  lane = lax.iota(jnp.int32, (L,))
  if heads == 8:
    rhalf = lane // 8
    col_ad = lane % 8
    n_grp = CHUNK // 2
    epg = 2
  else:
    rhalf = lane
    col_ad = jnp.zeros((L,), jnp.int32)
    n_grp = CHUNK // 16
    epg = 16
  col_as = col_ad + acol

  @pl.loop(0, CPT)
  def _(j):
    pltpu.async_copy(tab_hbm.at[sidx_v.at[j]], rows_v, sem_g).wait()
    pltpu.async_copy(adst_hbm.at[didx_v.at[j]], adst_v, sem_a).wait()

    @pl.loop(0, n_grp)
    def _(p):
      r0 = p * epg
      ridx = rhalf + r0
      asrc = plsc.load_gather(rows_v, [ridx, col_as])
      adst = plsc.load_gather(adst_v, [ridx, col_ad])
      al = asrc + adst
      al = jnp.where(al >= 0.0, al, al * 0.2)
      ev = jnp.exp(al)
      plsc.store_scatter(rows_v, [ridx, col_as], ev)
      for e in range(epg):
        row = r0 + e
        for h in range(heads):
          w = rows_v[row, acol + h]
          sl = pl.ds(h * hid, hid)
          rows_v[row, sl] = rows_v[row, sl] * w

    pltpu.sync_copy(rows_v, acc_sh.at[didx_v.at[j]], add=True)

  plsc.subcore_barrier()
  for b in range(rpt):
    base = (s * rpt + b) * CHUNK
    pltpu.sync_copy(acc_sh.at[pl.ds(base, CHUNK)],
                    out_hbm.at[c, pl.ds(base, CHUNK)])


def _make_edge_call(heads, hid, width, acol):
  body = functools.partial(_edge_kernel, heads, hid, width, acol)
  return pl.kernel(
      body,
      out_type=jax.ShapeDtypeStruct((NC, NP, width), jnp.float32),
      mesh=_mesh,
      scratch_types=[
          pltpu.VMEM((CPT, CHUNK), jnp.int32),
          pltpu.VMEM((CPT, CHUNK), jnp.int32),
          pltpu.VMEM((CHUNK, width), jnp.float32),
          pltpu.VMEM((CHUNK, 8), jnp.float32),
          pltpu.VMEM_SHARED((NP, width), jnp.float32),
          pltpu.SemaphoreType.DMA,
          pltpu.SemaphoreType.DMA,
      ],
  )


_edge1 = _make_edge_call(HEADS, HIDDEN, D1, 128)
_edge2 = _make_edge_call(1, OUT_FEATS, D2, 64)


def _tc1_body(x_ref, w_ref, a_ref, tab_ref, adst_ref):
  h = jnp.dot(x_ref[...], w_ref[...], preferred_element_type=jnp.float32)
  aa = jnp.dot(h, a_ref[...], preferred_element_type=jnp.float32)
  tab_ref[...] = jnp.concatenate(
      [h, aa[:, 0:8], jnp.zeros((BLK, 8), jnp.float32)], axis=1)
  adst_ref[...] = aa[:, 8:16]


_tc1 = pl.pallas_call(
    _tc1_body,
    grid=(NP // BLK,),
    in_specs=[
        pl.BlockSpec((BLK, IN_FEATS), lambda i: (i, 0)),
        pl.BlockSpec((IN_FEATS, IN_FEATS), lambda i: (0, 0)),
        pl.BlockSpec((IN_FEATS, 16), lambda i: (0, 0)),
    ],
    out_specs=[
        pl.BlockSpec((BLK, D1), lambda i: (i, 0)),
        pl.BlockSpec((BLK, 8), lambda i: (i, 0)),
    ],
    out_shape=[
        jax.ShapeDtypeStruct((NP, D1), jnp.float32),
        jax.ShapeDtypeStruct((NP, 8), jnp.float32),
    ],
)


def _tc2_body(acc_ref, b1_ref, rep_ref, w2_ref, a2_ref, tab2_ref, adst2_ref):
  a = acc_ref[0] + acc_ref[1]
  denb = jnp.dot(a[:, 128:136], rep_ref[...],
                 preferred_element_type=jnp.float32) + 1e-16
  out1 = a[:, 0:128] / denb + b1_ref[...]
  out1 = jnp.where(out1 > 0.0, out1, jnp.expm1(out1))
  h2 = jnp.dot(out1, w2_ref[...], preferred_element_type=jnp.float32)
  aa = jnp.dot(h2, a2_ref[...], preferred_element_type=jnp.float32)
  tab2_ref[...] = jnp.concatenate(
      [h2, aa[:, 0:1], jnp.zeros((BLK, 15), jnp.float32)], axis=1)
  adst2_ref[...] = jnp.concatenate(
      [aa[:, 1:2], jnp.zeros((BLK, 7), jnp.float32)], axis=1)


_tc2 = pl.pallas_call(
    _tc2_body,
    grid=(NP // BLK,),
    in_specs=[
        pl.BlockSpec((NC, BLK, D1), lambda i: (0, i, 0)),
        pl.BlockSpec((1, IN_FEATS), lambda i: (0, 0)),
        pl.BlockSpec((8, 128), lambda i: (0, 0)),
        pl.BlockSpec((IN_FEATS, OUT_FEATS), lambda i: (0, 0)),
        pl.BlockSpec((OUT_FEATS, 16), lambda i: (0, 0)),
    ],
    out_specs=[
        pl.BlockSpec((BLK, D2), lambda i: (i, 0)),
        pl.BlockSpec((BLK, 8), lambda i: (i, 0)),
    ],
    out_shape=[
        jax.ShapeDtypeStruct((NP, D2), jnp.float32),
        jax.ShapeDtypeStruct((NP, 8), jnp.float32),
    ],
)


def _tc3_body(acc_ref, b2_ref, out_ref):
  a = acc_ref[0] + acc_ref[1]
  denom = a[:, 64:65] + 1e-16
  out_ref[...] = a[:, 0:64] / denom + b2_ref[...]


_tc3 = pl.pallas_call(
    _tc3_body,
    grid=(NP // BLK,),
    in_specs=[
        pl.BlockSpec((NC, BLK, D2), lambda i: (0, i, 0)),
        pl.BlockSpec((1, OUT_FEATS), lambda i: (0, 0)),
    ],
    out_specs=pl.BlockSpec((BLK, OUT_FEATS), lambda i: (i, 0)),
    out_shape=jax.ShapeDtypeStruct((NP, OUT_FEATS), jnp.float32),
)


def kernel(x, edge_index, W1, att_src1, att_dst1, b1, W2, att_src2, att_dst2,
           b2):
  f32 = jnp.float32
  xp = jnp.zeros((NP, IN_FEATS), f32).at[:N].set(x)
  pad = E_PAD - E
  src_p = jnp.concatenate([edge_index[0], jnp.zeros((pad,), jnp.int32)])
  dst_p = jnp.concatenate([edge_index[1], jnp.full((pad,), N, jnp.int32)])
  sidx = src_p.reshape(NW, CPT, CHUNK)
  didx = dst_p.reshape(NW, CPT, CHUNK)

  eye_rep = jnp.repeat(jnp.eye(HEADS, dtype=f32), HIDDEN, axis=0)
  a1 = jnp.concatenate(
      [eye_rep * att_src1[0].reshape(-1, 1),
       eye_rep * att_dst1[0].reshape(-1, 1)], axis=1)
  rep = jnp.repeat(jnp.eye(HEADS, dtype=f32), HIDDEN, axis=1)
  a2 = jnp.zeros((OUT_FEATS, 16), f32)
  a2 = a2.at[:, 0].set(att_src2[0, 0]).at[:, 1].set(att_dst2[0, 0])

  tab1, adst1 = _tc1(xp, W1, a1)
  acc1 = _edge1(tab1, adst1, sidx, didx)
  tab2, adst2 = _tc2(acc1, b1.reshape(1, -1), rep, W2, a2)
  acc2 = _edge2(tab2, adst2, sidx, didx)
  out = _tc3(acc2, b2.reshape(1, -1))
  return out[:N]


# speedup vs baseline: 41.3149x; 41.3149x over previous
import dataclasses
import functools

import jax
import jax.numpy as jnp
from jax import lax
from jax.experimental import pallas as pl
from jax.experimental.pallas import tpu as pltpu
from jax.experimental.pallas import tpu_sc as plsc

N = 10000
E = 320000
IN_FEATS = 128
HIDDEN = 16
HEADS = 8
OUT_FEATS = 64

NC = 2
NS = 16
NW = NC * NS
L = 16

CHUNK = 128
CPT = -(-E // (NW * CHUNK))
E_PAD = NW * CPT * CHUNK

NP = 10048
RPS = NP // NS
D1 = 144
D2 = 80
BLK = 256

_mesh = plsc.VectorSubcoreMesh(core_axis_name="c", subcore_axis_name="s")


def _zero_rows(rows_v, width):
  z = jnp.zeros((L,), jnp.float32)

  @pl.loop(0, CHUNK)
  def _(r):
    for q in range(width // L):
      rows_v[r, pl.ds(q * L, L)] = z


def _edge_kernel(heads, hid, width, acol,
                 tab_hbm, adst_hbm, sidx_hbm, didx_hbm, out_hbm,
                 sidx_v, didx_v, rows_v, adst_v, acc_sh, sem_g, sem_a):
  c = lax.axis_index("c")
  s = lax.axis_index("s")
  wid = c * NS + s

  _zero_rows(rows_v, width)
  nfull = RPS // CHUNK
  rem = RPS - nfull * CHUNK
  for b in range(nfull):
    base = s * RPS + b * CHUNK
    pltpu.sync_copy(rows_v, acc_sh.at[pl.ds(base, CHUNK)])
  pltpu.sync_copy(rows_v.at[pl.ds(0, rem)],
                  acc_sh.at[pl.ds(s * RPS + nfull * CHUNK, rem)])
  plsc.subcore_barrier()

  pltpu.sync_copy(sidx_hbm.at[wid], sidx_v)
  pltpu.sync_copy(didx_hbm.at[wid], didx_v)

  lane = lax.iota(jnp.int32, L)
  if heads == 8:
    rhalf = lane // 8
    col_ad = lane % 8
    n_grp = CHUNK // 2
    epg = 2
  else:
    rhalf = lane
    col_ad = jnp.zeros((L,), jnp.int32)
    n_grp = CHUNK // 16
    epg = 16
  col_as = col_ad + acol

  @pl.loop(0, CPT)
  def _(j):
    pltpu.async_copy(tab_hbm.at[sidx_v.at[j]], rows_v, sem_g).wait()
    pltpu.async_copy(adst_hbm.at[didx_v.at[j]], adst_v, sem_a).wait()

    @pl.loop(0, n_grp)
    def _(p):
      r0 = p * epg
      ridx = rhalf + r0
      asrc = plsc.load_gather(rows_v, [ridx, col_as])
      adst = plsc.load_gather(adst_v, [ridx, col_ad])
      al = asrc + adst
      al = jnp.where(al >= 0.0, al, al * 0.2)
      ev = jnp.exp(al)
      plsc.store_scatter(rows_v, [ridx, col_as], ev)
      for e in range(epg):
        row = r0 + e
        wv = rows_v[row, pl.ds(acol, L)]
        for h in range(heads):
          w = wv[h]
          for q in range(hid // L):
            sl = pl.ds(h * hid + q * L, L)
            rows_v[row, sl] = rows_v[row, sl] * w

    pltpu.sync_copy(rows_v, acc_sh.at[didx_v.at[j]], add=True)

  plsc.subcore_barrier()
  for b in range(nfull):
    base = s * RPS + b * CHUNK
    pltpu.sync_copy(acc_sh.at[pl.ds(base, CHUNK)],
                    out_hbm.at[c, pl.ds(base, CHUNK)])
  tail = s * RPS + nfull * CHUNK
  pltpu.sync_copy(acc_sh.at[pl.ds(tail, rem)],
                  out_hbm.at[c, pl.ds(tail, rem)])


_sc_params = pltpu.CompilerParams(
    needs_layout_passes=False, use_tc_tiling_on_sc=False)


def _make_edge_call(heads, hid, width, acol):
  body = functools.partial(_edge_kernel, heads, hid, width, acol)
  return pl.kernel(
      body,
      compiler_params=_sc_params,
      out_type=jax.ShapeDtypeStruct((NC, NP, width), jnp.float32),
      mesh=_mesh,
      scratch_types=[
          pltpu.VMEM((CPT, CHUNK), jnp.int32),
          pltpu.VMEM((CPT, CHUNK), jnp.int32),
          pltpu.VMEM((CHUNK, width), jnp.float32),
          pltpu.VMEM((CHUNK, 8), jnp.float32),
          pltpu.VMEM_SHARED((NP, width), jnp.float32),
          pltpu.SemaphoreType.DMA,
          pltpu.SemaphoreType.DMA,
      ],
  )


_edge1 = _make_edge_call(HEADS, HIDDEN, D1, 128)
_edge2 = _make_edge_call(1, OUT_FEATS, D2, 64)


def _tc1_body(x_ref, w_ref, a_ref, tab_ref, adst_ref):
  h = jnp.dot(x_ref[...], w_ref[...], preferred_element_type=jnp.float32)
  aa = jnp.dot(h, a_ref[...], preferred_element_type=jnp.float32)
  tab_ref[...] = jnp.concatenate(
      [h, aa[:, 0:8], jnp.zeros((BLK, 8), jnp.float32)], axis=1)
  adst_ref[...] = aa[:, 8:16]


_tc1 = pl.pallas_call(
    _tc1_body,
    grid=(-(-NP // BLK),),
    in_specs=[
        pl.BlockSpec((BLK, IN_FEATS), lambda i: (i, 0)),
        pl.BlockSpec((IN_FEATS, IN_FEATS), lambda i: (0, 0)),
        pl.BlockSpec((IN_FEATS, 16), lambda i: (0, 0)),
    ],
    out_specs=[
        pl.BlockSpec((BLK, D1), lambda i: (i, 0)),
        pl.BlockSpec((BLK, 8), lambda i: (i, 0)),
    ],
    out_shape=[
        jax.ShapeDtypeStruct((NP, D1), jnp.float32),
        jax.ShapeDtypeStruct((NP, 8), jnp.float32),
    ],
)


def _tc2_body(acc_ref, b1_ref, rep_ref, w2_ref, a2_ref, tab2_ref, adst2_ref):
  a = acc_ref[0] + acc_ref[1]
  denb = jnp.dot(a[:, 128:136], rep_ref[...],
                 preferred_element_type=jnp.float32) + 1e-16
  out1 = a[:, 0:128] / denb + b1_ref[...]
  out1 = jnp.where(out1 > 0.0, out1, jnp.exp(out1) - 1.0)
  h2 = jnp.dot(out1, w2_ref[...], preferred_element_type=jnp.float32)
  aa = jnp.dot(h2, a2_ref[...], preferred_element_type=jnp.float32)
  tab2_ref[...] = jnp.concatenate(
      [h2, aa[:, 0:1], jnp.zeros((BLK, 15), jnp.float32)], axis=1)
  adst2_ref[...] = jnp.concatenate(
      [aa[:, 1:2], jnp.zeros((BLK, 7), jnp.float32)], axis=1)


_tc2 = pl.pallas_call(
    _tc2_body,
    grid=(-(-NP // BLK),),
    in_specs=[
        pl.BlockSpec((NC, BLK, D1), lambda i: (0, i, 0)),
        pl.BlockSpec((1, IN_FEATS), lambda i: (0, 0)),
        pl.BlockSpec((8, 128), lambda i: (0, 0)),
        pl.BlockSpec((IN_FEATS, OUT_FEATS), lambda i: (0, 0)),
        pl.BlockSpec((OUT_FEATS, 16), lambda i: (0, 0)),
    ],
    out_specs=[
        pl.BlockSpec((BLK, D2), lambda i: (i, 0)),
        pl.BlockSpec((BLK, 8), lambda i: (i, 0)),
    ],
    out_shape=[
        jax.ShapeDtypeStruct((NP, D2), jnp.float32),
        jax.ShapeDtypeStruct((NP, 8), jnp.float32),
    ],
)


def _tc3_body(acc_ref, b2_ref, out_ref):
  a = acc_ref[0] + acc_ref[1]
  denom = a[:, 64:65] + 1e-16
  out_ref[...] = a[:, 0:64] / denom + b2_ref[...]


_tc3 = pl.pallas_call(
    _tc3_body,
    grid=(-(-NP // BLK),),
    in_specs=[
        pl.BlockSpec((NC, BLK, D2), lambda i: (0, i, 0)),
        pl.BlockSpec((1, OUT_FEATS), lambda i: (0, 0)),
    ],
    out_specs=pl.BlockSpec((BLK, OUT_FEATS), lambda i: (i, 0)),
    out_shape=jax.ShapeDtypeStruct((NP, OUT_FEATS), jnp.float32),
)


def kernel(x, edge_index, W1, att_src1, att_dst1, b1, W2, att_src2, att_dst2,
           b2):
  f32 = jnp.float32
  xp = jnp.zeros((NP, IN_FEATS), f32).at[:N].set(x)
  pad = E_PAD - E
  src_p = jnp.concatenate([edge_index[0], jnp.zeros((pad,), jnp.int32)])
  dst_p = jnp.concatenate([edge_index[1], jnp.full((pad,), N, jnp.int32)])
  sidx = src_p.reshape(NW, CPT, CHUNK)
  didx = dst_p.reshape(NW, CPT, CHUNK)

  eye_rep = jnp.repeat(jnp.eye(HEADS, dtype=f32), HIDDEN, axis=0)
  a1 = jnp.concatenate(
      [eye_rep * att_src1[0].reshape(-1, 1),
       eye_rep * att_dst1[0].reshape(-1, 1)], axis=1)
  rep = jnp.repeat(jnp.eye(HEADS, dtype=f32), HIDDEN, axis=1)
  a2 = jnp.zeros((OUT_FEATS, 16), f32)
  a2 = a2.at[:, 0].set(att_src2[0, 0]).at[:, 1].set(att_dst2[0, 0])

  tab1, adst1 = _tc1(xp, W1, a1)
  acc1 = _edge1(tab1, adst1, sidx, didx)
  tab2, adst2 = _tc2(acc1, b1.reshape(1, -1), rep, W2, a2)
  acc2 = _edge2(tab2, adst2, sidx, didx)
  out = _tc3(acc2, b2.reshape(1, -1))
  return out[:N]


# speedup vs baseline: 48.9141x; 1.1839x over previous
import dataclasses
import functools

import jax
import jax.numpy as jnp
from jax import lax
from jax.experimental import pallas as pl
from jax.experimental.pallas import tpu as pltpu
from jax.experimental.pallas import tpu_sc as plsc

N = 10000
E = 320000
IN_FEATS = 128
HIDDEN = 16
HEADS = 8
OUT_FEATS = 64

NC = 2
NS = 16
NW = NC * NS
L = 16

CHUNK = 128
NBUF = 2
IGRP = 2
CPT = 80
E_PAD = NW * CPT * CHUNK

NP = 10016
RPS = NP // NS
D1 = 144
D2 = 80
BLK = 256

_mesh = plsc.VectorSubcoreMesh(core_axis_name="c", subcore_axis_name="s")


def _zero_rows(rows_v, width):
  z = jnp.zeros((L,), jnp.float32)

  @pl.loop(0, CHUNK)
  def _(r):
    for q in range(width // L):
      rows_v[r, pl.ds(q * L, L)] = z


def _edge_kernel(heads, hid, width, acol,
                 tab_hbm, adst_hbm, sidx_hbm, didx_hbm, out_hbm,
                 sidx_v, didx_v, rows_v, adst_v, acc_sh, gsem, ssem):
  c = lax.axis_index("c")
  s = lax.axis_index("s")
  wid = c * NS + s

  _zero_rows(rows_v.at[0], width)
  nfull = RPS // CHUNK
  rem = RPS - nfull * CHUNK
  for b in range(nfull):
    base = s * RPS + b * CHUNK
    pltpu.sync_copy(rows_v.at[0], acc_sh.at[pl.ds(base, CHUNK)])
  pltpu.sync_copy(rows_v.at[0].at[pl.ds(0, rem)],
                  acc_sh.at[pl.ds(s * RPS + nfull * CHUNK, rem)])
  plsc.subcore_barrier()

  lane = lax.iota(jnp.int32, L)
  if heads == 8:
    rhalf = lane // 8
    col_ad = lane % 8
    n_grp = CHUNK // 2
    epg = 2
  else:
    rhalf = lane
    col_ad = jnp.zeros((L,), jnp.int32)
    n_grp = CHUNK // 16
    epg = 16
  col_as = col_ad + acol

  def _issue_gather(i, g, b):
    pltpu.async_copy(tab_hbm.at[sidx_v.at[g, i % IGRP]], rows_v.at[b],
                     gsem.at[b])
    pltpu.async_copy(adst_hbm.at[didx_v.at[g, i % IGRP]], adst_v.at[b],
                     gsem.at[b])

  def _wait_gather(i, g, b):
    pltpu.make_async_copy(tab_hbm.at[sidx_v.at[g, i % IGRP]], rows_v.at[b],
                          gsem.at[b]).wait()
    pltpu.make_async_copy(adst_hbm.at[didx_v.at[g, i % IGRP]], adst_v.at[b],
                          gsem.at[b]).wait()

  def _drain_scatter(b):
    pltpu.make_async_copy(rows_v.at[b], acc_sh.at[didx_v.at[0, 0]],
                          ssem.at[b]).wait()

  def _stage_idx(first_chunk, g):
    pltpu.sync_copy(sidx_hbm.at[wid, pl.ds(first_chunk, IGRP)], sidx_v.at[g])
    pltpu.sync_copy(didx_hbm.at[wid, pl.ds(first_chunk, IGRP)], didx_v.at[g])

  _stage_idx(0, 0)
  _issue_gather(0, 0, 0)

  @pl.loop(0, CPT, step=NBUF)
  def _(j):
    for b in range(NBUF):
      i = j + b
      b1 = 1 - b
      g = (i // IGRP) % 2

      @pl.when(i >= 1)
      def _():
        _drain_scatter(b1)

      if b == IGRP - 1:
        @pl.when(i + 1 < CPT)
        def _():
          _stage_idx(i + 1, (1 - g) if IGRP == NBUF else ((i + 1) // IGRP) % 2)

      g1 = ((i + 1) // IGRP) % 2

      @pl.when(i + 1 < CPT)
      def _():
        _issue_gather(i + 1, g1, b1)

      _wait_gather(i, g, b)
      rows = rows_v.at[b]
      adst_b = adst_v.at[b]

      @pl.loop(0, n_grp)
      def _(p):
        r0 = p * epg
        ridx = rhalf + r0
        asrc = plsc.load_gather(rows, [ridx, col_as])
        adst = plsc.load_gather(adst_b, [ridx, col_ad])
        al = asrc + adst
        al = jnp.where(al >= 0.0, al, al * 0.2)
        ev = jnp.exp(al)
        plsc.store_scatter(rows, [ridx, col_as], ev)
        for e in range(epg):
          row = r0 + e
          wv = rows[row, pl.ds(acol, L)]
          for h in range(heads):
            w = wv[h]
            for q in range(hid // L):
              sl = pl.ds(h * hid + q * L, L)
              rows[row, sl] = rows[row, sl] * w

      pltpu.async_copy(rows_v.at[b], acc_sh.at[didx_v.at[g, i % IGRP]],
                       ssem.at[b], add=True)

  _drain_scatter((CPT - 1) % NBUF)

  plsc.subcore_barrier()
  for b in range(nfull):
    base = s * RPS + b * CHUNK
    pltpu.sync_copy(acc_sh.at[pl.ds(base, CHUNK)],
                    out_hbm.at[c, pl.ds(base, CHUNK)])
  tail = s * RPS + nfull * CHUNK
  pltpu.sync_copy(acc_sh.at[pl.ds(tail, rem)],
                  out_hbm.at[c, pl.ds(tail, rem)])


_sc_params = pltpu.CompilerParams(
    needs_layout_passes=False, use_tc_tiling_on_sc=False)


def _make_edge_call(heads, hid, width, acol):
  body = functools.partial(_edge_kernel, heads, hid, width, acol)
  return pl.kernel(
      body,
      compiler_params=_sc_params,
      out_type=jax.ShapeDtypeStruct((NC, NP, width), jnp.float32),
      mesh=_mesh,
      scratch_types=[
          pltpu.VMEM((2, IGRP, CHUNK), jnp.int32),
          pltpu.VMEM((2, IGRP, CHUNK), jnp.int32),
          pltpu.VMEM((NBUF, CHUNK, width), jnp.float32),
          pltpu.VMEM((NBUF, CHUNK, 8), jnp.float32),
          pltpu.VMEM_SHARED((NP, width), jnp.float32),
          pltpu.SemaphoreType.DMA((NBUF,)),
          pltpu.SemaphoreType.DMA((NBUF,)),
      ],
  )


_edge1 = _make_edge_call(HEADS, HIDDEN, D1, 128)
_edge2 = _make_edge_call(1, OUT_FEATS, D2, 64)


def _tc1_body(x_ref, w_ref, a_ref, tab_ref, adst_ref):
  h = jnp.dot(x_ref[...], w_ref[...], preferred_element_type=jnp.float32)
  aa = jnp.dot(h, a_ref[...], preferred_element_type=jnp.float32)
  tab_ref[...] = jnp.concatenate(
      [h, aa[:, 0:8], jnp.zeros((BLK, 8), jnp.float32)], axis=1)
  adst_ref[...] = aa[:, 8:16]


_tc1 = pl.pallas_call(
    _tc1_body,
    grid=(-(-NP // BLK),),
    in_specs=[
        pl.BlockSpec((BLK, IN_FEATS), lambda i: (i, 0)),
        pl.BlockSpec((IN_FEATS, IN_FEATS), lambda i: (0, 0)),
        pl.BlockSpec((IN_FEATS, 16), lambda i: (0, 0)),
    ],
    out_specs=[
        pl.BlockSpec((BLK, D1), lambda i: (i, 0)),
        pl.BlockSpec((BLK, 8), lambda i: (i, 0)),
    ],
    out_shape=[
        jax.ShapeDtypeStruct((NP, D1), jnp.float32),
        jax.ShapeDtypeStruct((NP, 8), jnp.float32),
    ],
)


def _tc2_body(acc_ref, b1_ref, rep_ref, w2_ref, a2_ref, tab2_ref, adst2_ref):
  a = acc_ref[0] + acc_ref[1]
  denb = jnp.dot(a[:, 128:136], rep_ref[...],
                 preferred_element_type=jnp.float32) + 1e-16
  out1 = a[:, 0:128] / denb + b1_ref[...]
  out1 = jnp.where(out1 > 0.0, out1, jnp.exp(out1) - 1.0)
  h2 = jnp.dot(out1, w2_ref[...], preferred_element_type=jnp.float32)
  aa = jnp.dot(h2, a2_ref[...], preferred_element_type=jnp.float32)
  tab2_ref[...] = jnp.concatenate(
      [h2, aa[:, 0:1], jnp.zeros((BLK, 15), jnp.float32)], axis=1)
  adst2_ref[...] = jnp.concatenate(
      [aa[:, 1:2], jnp.zeros((BLK, 7), jnp.float32)], axis=1)


_tc2 = pl.pallas_call(
    _tc2_body,
    grid=(-(-NP // BLK),),
    in_specs=[
        pl.BlockSpec((NC, BLK, D1), lambda i: (0, i, 0)),
        pl.BlockSpec((1, IN_FEATS), lambda i: (0, 0)),
        pl.BlockSpec((8, 128), lambda i: (0, 0)),
        pl.BlockSpec((IN_FEATS, OUT_FEATS), lambda i: (0, 0)),
        pl.BlockSpec((OUT_FEATS, 16), lambda i: (0, 0)),
    ],
    out_specs=[
        pl.BlockSpec((BLK, D2), lambda i: (i, 0)),
        pl.BlockSpec((BLK, 8), lambda i: (i, 0)),
    ],
    out_shape=[
        jax.ShapeDtypeStruct((NP, D2), jnp.float32),
        jax.ShapeDtypeStruct((NP, 8), jnp.float32),
    ],
)


def _tc3_body(acc_ref, b2_ref, out_ref):
  a = acc_ref[0] + acc_ref[1]
  denom = a[:, 64:65] + 1e-16
  out_ref[...] = a[:, 0:64] / denom + b2_ref[...]


_tc3 = pl.pallas_call(
    _tc3_body,
    grid=(-(-NP // BLK),),
    in_specs=[
        pl.BlockSpec((NC, BLK, D2), lambda i: (0, i, 0)),
        pl.BlockSpec((1, OUT_FEATS), lambda i: (0, 0)),
    ],
    out_specs=pl.BlockSpec((BLK, OUT_FEATS), lambda i: (i, 0)),
    out_shape=jax.ShapeDtypeStruct((NP, OUT_FEATS), jnp.float32),
)


def kernel(x, edge_index, W1, att_src1, att_dst1, b1, W2, att_src2, att_dst2,
           b2):
  f32 = jnp.float32
  xp = jnp.zeros((NP, IN_FEATS), f32).at[:N].set(x)
  pad = E_PAD - E
  src_p = jnp.concatenate([edge_index[0], jnp.zeros((pad,), jnp.int32)])
  dst_p = jnp.concatenate([edge_index[1], jnp.full((pad,), N, jnp.int32)])
  sidx = src_p.reshape(NW, CPT, CHUNK)
  didx = dst_p.reshape(NW, CPT, CHUNK)

  eye_rep = jnp.repeat(jnp.eye(HEADS, dtype=f32), HIDDEN, axis=0)
  a1 = jnp.concatenate(
      [eye_rep * att_src1[0].reshape(-1, 1),
       eye_rep * att_dst1[0].reshape(-1, 1)], axis=1)
  rep = jnp.repeat(jnp.eye(HEADS, dtype=f32), HIDDEN, axis=1)
  a2 = jnp.zeros((OUT_FEATS, 16), f32)
  a2 = a2.at[:, 0].set(att_src2[0, 0]).at[:, 1].set(att_dst2[0, 0])

  tab1, adst1 = _tc1(xp, W1, a1)
  acc1 = _edge1(tab1, adst1, sidx, didx)
  tab2, adst2 = _tc2(acc1, b1.reshape(1, -1), rep, W2, a2)
  acc2 = _edge2(tab2, adst2, sidx, didx)
  out = _tc3(acc2, b2.reshape(1, -1))
  return out[:N]


# speedup vs baseline: 54.4023x; 1.1122x over previous
import functools

import jax
import jax.numpy as jnp
from jax import lax
from jax.experimental import pallas as pl
from jax.experimental.pallas import tpu as pltpu
from jax.experimental.pallas import tpu_sc as plsc

N = 10000
E = 320000
IN_FEATS = 128
HIDDEN = 16
HEADS = 8
OUT_FEATS = 64

NC = 2
NS = 16
NW = NC * NS
L = 16

CHUNK = 128
NBUF = 2
IGRP = 2
NCHUNKS = 2560
E_PAD = NCHUNKS * CHUNK

NP = 10016
RPS = NP // NS
D1 = 144
D2 = 80
BLK = 256

_mesh = plsc.VectorSubcoreMesh(core_axis_name="c", subcore_axis_name="s")


def _zero_rows(rows_v, width):
  z = jnp.zeros((L,), jnp.float32)

  @pl.loop(0, CHUNK)
  def _(r):
    for q in range(width // L):
      rows_v[r, pl.ds(q * L, L)] = z


def _edge_kernel(heads, hid, width, acol, s0, s1,
                 tab_hbm, adst_hbm, sidx_hbm, didx_hbm, out_hbm,
                 sidx_v, didx_v, rows_v, adst_v, acc_sh, gsem, ssem):
  c = lax.axis_index("c")
  s = lax.axis_index("s")
  base_t = jnp.where(c == 0, s * s0, NS * s0 + s * s1)
  cnt = jnp.where(c == 0, s0, s1)

  _zero_rows(rows_v.at[0], width)
  nfull = RPS // CHUNK
  rem = RPS - nfull * CHUNK
  for b in range(nfull):
    base = s * RPS + b * CHUNK
    pltpu.sync_copy(rows_v.at[0], acc_sh.at[pl.ds(base, CHUNK)])
  pltpu.sync_copy(rows_v.at[0].at[pl.ds(0, rem)],
                  acc_sh.at[pl.ds(s * RPS + nfull * CHUNK, rem)])
  plsc.subcore_barrier()

  lane = lax.iota(jnp.int32, L)
  if heads == 8:
    rhalf = lane // 8
    col_ad = lane % 8
    n_grp = CHUNK // 2
    epg = 2
  else:
    rhalf = lane
    col_ad = jnp.zeros((L,), jnp.int32)
    n_grp = CHUNK // 16
    epg = 16
  col_as = col_ad + acol

  def _issue_gather(i, g, b):
    pltpu.async_copy(tab_hbm.at[sidx_v.at[g, i % IGRP]], rows_v.at[b],
                     gsem.at[b])
    pltpu.async_copy(adst_hbm.at[didx_v.at[g, i % IGRP]], adst_v.at[b],
                     gsem.at[b])

  def _wait_gather(i, g, b):
    pltpu.make_async_copy(tab_hbm.at[sidx_v.at[g, i % IGRP]], rows_v.at[b],
                          gsem.at[b]).wait()
    pltpu.make_async_copy(adst_hbm.at[didx_v.at[g, i % IGRP]], adst_v.at[b],
                          gsem.at[b]).wait()

  def _drain_scatter(b):
    pltpu.make_async_copy(rows_v.at[b], acc_sh.at[didx_v.at[0, 0]],
                          ssem.at[b]).wait()

  def _stage_idx(first_chunk, g):
    pltpu.sync_copy(sidx_hbm.at[pl.ds(base_t + first_chunk, IGRP)],
                    sidx_v.at[g])
    pltpu.sync_copy(didx_hbm.at[pl.ds(base_t + first_chunk, IGRP)],
                    didx_v.at[g])

  _stage_idx(0, 0)
  _issue_gather(0, 0, 0)

  @pl.loop(0, cnt, step=NBUF)
  def _(j):
    for b in range(NBUF):
      i = j + b
      b1 = 1 - b
      g = (i // IGRP) % 2

      @pl.when(i >= 1)
      def _():
        _drain_scatter(b1)

      if b == IGRP - 1:
        @pl.when(i + 1 < cnt)
        def _():
          _stage_idx(i + 1, (1 - g) if IGRP == NBUF else ((i + 1) // IGRP) % 2)

      g1 = ((i + 1) // IGRP) % 2

      @pl.when(i + 1 < cnt)
      def _():
        _issue_gather(i + 1, g1, b1)

      _wait_gather(i, g, b)
      rows = rows_v.at[b]
      adst_b = adst_v.at[b]

      @pl.loop(0, n_grp)
      def _(p):
        r0 = p * epg
        ridx = rhalf + r0
        asrc = plsc.load_gather(rows, [ridx, col_as])
        adst = plsc.load_gather(adst_b, [ridx, col_ad])
        al = asrc + adst
        al = jnp.where(al >= 0.0, al, al * 0.2)
        ev = jnp.exp(al)
        plsc.store_scatter(rows, [ridx, col_as], ev)
        for e in range(epg):
          row = r0 + e
          wv = rows[row, pl.ds(acol, L)]
          for h in range(heads):
            w = wv[h]
            for q in range(hid // L):
              sl = pl.ds(h * hid + q * L, L)
              rows[row, sl] = rows[row, sl] * w

      pltpu.async_copy(rows_v.at[b], acc_sh.at[didx_v.at[g, i % IGRP]],
                       ssem.at[b], add=True)

  _drain_scatter(1)

  plsc.subcore_barrier()
  for b in range(nfull):
    base = s * RPS + b * CHUNK
    pltpu.sync_copy(acc_sh.at[pl.ds(base, CHUNK)],
                    out_hbm.at[c, pl.ds(base, CHUNK)])
  tail = s * RPS + nfull * CHUNK
  pltpu.sync_copy(acc_sh.at[pl.ds(tail, rem)],
                  out_hbm.at[c, pl.ds(tail, rem)])


_sc_params = pltpu.CompilerParams(
    needs_layout_passes=False, use_tc_tiling_on_sc=False)


def _make_edge_call(heads, hid, width, acol, s0, s1):
  assert s0 % NBUF == 0 and s1 % NBUF == 0 and NS * (s0 + s1) == NCHUNKS
  body = functools.partial(_edge_kernel, heads, hid, width, acol, s0, s1)
  return pl.kernel(
      body,
      compiler_params=_sc_params,
      out_type=jax.ShapeDtypeStruct((NC, NP, width), jnp.float32),
      mesh=_mesh,
      scratch_types=[
          pltpu.VMEM((2, IGRP, CHUNK), jnp.int32),
          pltpu.VMEM((2, IGRP, CHUNK), jnp.int32),
          pltpu.VMEM((NBUF, CHUNK, width), jnp.float32),
          pltpu.VMEM((NBUF, CHUNK, 8), jnp.float32),
          pltpu.VMEM_SHARED((NP, width), jnp.float32),
          pltpu.SemaphoreType.DMA((NBUF,)),
          pltpu.SemaphoreType.DMA((NBUF,)),
      ],
  )


_edge1 = _make_edge_call(HEADS, HIDDEN, D1, 128, 112, 48)
_edge2 = _make_edge_call(1, OUT_FEATS, D2, 64, 92, 68)


def _tc1_body(x_ref, w_ref, a_ref, tab_ref, adst_ref):
  h = jnp.dot(x_ref[...], w_ref[...], preferred_element_type=jnp.float32)
  aa = jnp.dot(h, a_ref[...], preferred_element_type=jnp.float32)
  tab_ref[...] = jnp.concatenate(
      [h, aa[:, 0:8], jnp.zeros((BLK, 8), jnp.float32)], axis=1)
  adst_ref[...] = aa[:, 8:16]


_tc1 = pl.pallas_call(
    _tc1_body,
    grid=(-(-NP // BLK),),
    in_specs=[
        pl.BlockSpec((BLK, IN_FEATS), lambda i: (i, 0)),
        pl.BlockSpec((IN_FEATS, IN_FEATS), lambda i: (0, 0)),
        pl.BlockSpec((IN_FEATS, 16), lambda i: (0, 0)),
    ],
    out_specs=[
        pl.BlockSpec((BLK, D1), lambda i: (i, 0)),
        pl.BlockSpec((BLK, 8), lambda i: (i, 0)),
    ],
    out_shape=[
        jax.ShapeDtypeStruct((NP, D1), jnp.float32),
        jax.ShapeDtypeStruct((NP, 8), jnp.float32),
    ],
)


def _tc2_body(acc_ref, b1_ref, rep_ref, w2_ref, a2_ref, tab2_ref, adst2_ref):
  a = acc_ref[0] + acc_ref[1]
  denb = jnp.dot(a[:, 128:136], rep_ref[...],
                 preferred_element_type=jnp.float32) + 1e-16
  out1 = a[:, 0:128] / denb + b1_ref[...]
  out1 = jnp.where(out1 > 0.0, out1, jnp.exp(out1) - 1.0)
  h2 = jnp.dot(out1, w2_ref[...], preferred_element_type=jnp.float32)
  aa = jnp.dot(h2, a2_ref[...], preferred_element_type=jnp.float32)
  tab2_ref[...] = jnp.concatenate(
      [h2, aa[:, 0:1], jnp.zeros((BLK, 15), jnp.float32)], axis=1)
  adst2_ref[...] = jnp.concatenate(
      [aa[:, 1:2], jnp.zeros((BLK, 7), jnp.float32)], axis=1)


_tc2 = pl.pallas_call(
    _tc2_body,
    grid=(-(-NP // BLK),),
    in_specs=[
        pl.BlockSpec((NC, BLK, D1), lambda i: (0, i, 0)),
        pl.BlockSpec((1, IN_FEATS), lambda i: (0, 0)),
        pl.BlockSpec((8, 128), lambda i: (0, 0)),
        pl.BlockSpec((IN_FEATS, OUT_FEATS), lambda i: (0, 0)),
        pl.BlockSpec((OUT_FEATS, 16), lambda i: (0, 0)),
    ],
    out_specs=[
        pl.BlockSpec((BLK, D2), lambda i: (i, 0)),
        pl.BlockSpec((BLK, 8), lambda i: (i, 0)),
    ],
    out_shape=[
        jax.ShapeDtypeStruct((NP, D2), jnp.float32),
        jax.ShapeDtypeStruct((NP, 8), jnp.float32),
    ],
)


def _tc3_body(acc_ref, b2_ref, out_ref):
  a = acc_ref[0] + acc_ref[1]
  denom = a[:, 64:65] + 1e-16
  out_ref[...] = a[:, 0:64] / denom + b2_ref[...]


_tc3 = pl.pallas_call(
    _tc3_body,
    grid=(-(-NP // BLK),),
    in_specs=[
        pl.BlockSpec((NC, BLK, D2), lambda i: (0, i, 0)),
        pl.BlockSpec((1, OUT_FEATS), lambda i: (0, 0)),
    ],
    out_specs=pl.BlockSpec((BLK, OUT_FEATS), lambda i: (i, 0)),
    out_shape=jax.ShapeDtypeStruct((NP, OUT_FEATS), jnp.float32),
)


def kernel(x, edge_index, W1, att_src1, att_dst1, b1, W2, att_src2, att_dst2,
           b2):
  f32 = jnp.float32
  xp = jnp.zeros((NP, IN_FEATS), f32).at[:N].set(x)
  pad = E_PAD - E
  src_p = jnp.concatenate([edge_index[0], jnp.zeros((pad,), jnp.int32)])
  dst_p = jnp.concatenate([edge_index[1], jnp.full((pad,), N, jnp.int32)])
  sidx = src_p.reshape(NCHUNKS, CHUNK)
  didx = dst_p.reshape(NCHUNKS, CHUNK)

  eye_rep = jnp.repeat(jnp.eye(HEADS, dtype=f32), HIDDEN, axis=0)
  a1 = jnp.concatenate(
      [eye_rep * att_src1[0].reshape(-1, 1),
       eye_rep * att_dst1[0].reshape(-1, 1)], axis=1)
  rep = jnp.repeat(jnp.eye(HEADS, dtype=f32), HIDDEN, axis=1)
  a2 = jnp.zeros((OUT_FEATS, 16), f32)
  a2 = a2.at[:, 0].set(att_src2[0, 0]).at[:, 1].set(att_dst2[0, 0])

  tab1, adst1 = _tc1(xp, W1, a1)
  acc1 = _edge1(tab1, adst1, sidx, didx)
  tab2, adst2 = _tc2(acc1, b1.reshape(1, -1), rep, W2, a2)
  acc2 = _edge2(tab2, adst2, sidx, didx)
  out = _tc3(acc2, b2.reshape(1, -1))
  return out[:N]
